# baseline (speedup 1.0000x reference)
import functools

import jax
import jax.numpy as jnp
import numpy as np
from jax import lax
from jax.experimental import pallas as pl
from jax.experimental.pallas import tpu as pltpu
from jax.experimental.pallas import tpu_sc as plsc

N_NODES = 100000
DIM = 16
N_TICKS = 20
BATCH = 16384

_info = plsc.get_sparse_core_info()
NC, NS, L = _info.num_cores, _info.num_subcores, _info.num_lanes
NW = NC * NS
BW = BATCH // NW
CHUNK = 128
NCHUNK = BW // CHUNK
GROUPS = CHUNK // L

STEP = np.float32(1.0 / (N_TICKS - 1))
ROW = DIM * N_TICKS


def _body(src_h, dst_h, t_h, z_h, bias_h, out_h,
          src_v, dst_v, t_v, bias_v, srows, drows, out_v, sem):
    wid = lax.axis_index("s") * NC + lax.axis_index("c")
    pltpu.sync_copy(src_h.at[wid], src_v)
    pltpu.sync_copy(dst_h.at[wid], dst_v)
    pltpu.sync_copy(t_h.at[wid], t_v)
    pltpu.sync_copy(bias_h, bias_v)
    bias_vec = bias_v[...]
    iota = lax.iota(jnp.int32, L)

    for c in range(NCHUNK):
        cp_s = pltpu.async_copy(z_h.at[src_v.at[c]], srows, sem)
        cp_d = pltpu.async_copy(z_h.at[dst_v.at[c]], drows, sem)
        cp_s.wait()
        cp_d.wait()

        def group(g, carry, c=c):
            base = c * CHUNK + g * L
            tv = t_v[pl.ds(base, L)]
            q = tv / STEP
            ti = jnp.minimum(q.astype(jnp.int32), N_TICKS - 2)
            dt = lax.rem(tv, STEP) / STEP
            omdt = 1.0 - dt
            row = iota + g * L
            acc = jnp.zeros((L,), jnp.float32)
            for d in range(DIM):
                colc = ti + (d * N_TICKS)
                coln = colc + 1
                s_cur = plsc.load_gather(srows, [row, colc])
                s_nxt = plsc.load_gather(srows, [row, coln])
                d_cur = plsc.load_gather(drows, [row, colc])
                d_nxt = plsc.load_gather(drows, [row, coln])
                zs = omdt * s_cur + dt * s_nxt
                zd = omdt * d_cur + dt * d_nxt
                df = zs - zd
                acc = acc + df * df
            out_v[pl.ds(base, L)] = bias_vec - acc
            return carry

        lax.fori_loop(0, GROUPS, group, 0)

    pltpu.sync_copy(out_v, out_h.at[wid])


@functools.partial(
    pl.kernel,
    mesh=plsc.VectorSubcoreMesh(core_axis_name="c", subcore_axis_name="s"),
    out_type=jax.ShapeDtypeStruct((NW, BW), jnp.float32),
    compiler_params=pltpu.CompilerParams(
        use_tc_tiling_on_sc=False, needs_layout_passes=False),
    scratch_types=[
        pltpu.VMEM((NCHUNK, CHUNK), jnp.int32),
        pltpu.VMEM((NCHUNK, CHUNK), jnp.int32),
        pltpu.VMEM((BW,), jnp.float32),
        pltpu.VMEM((L,), jnp.float32),
        pltpu.VMEM((CHUNK, ROW), jnp.float32),
        pltpu.VMEM((CHUNK, ROW), jnp.float32),
        pltpu.VMEM((BW,), jnp.float32),
        pltpu.SemaphoreType.DMA,
    ],
)
def _decode_kernel(src_h, dst_h, t_h, z_h, bias_h, out_h, *scratch):
    _body(src_h, dst_h, t_h, z_h, bias_h, out_h, *scratch)


def kernel(src, dst, t, z, bias):
    src3 = src.astype(jnp.int32).reshape(NW, NCHUNK, CHUNK)
    dst3 = dst.astype(jnp.int32).reshape(NW, NCHUNK, CHUNK)
    t2 = t.reshape(NW, BW)
    z2 = z.reshape(N_NODES, ROW)
    bias_vec = jnp.full((L,), bias, dtype=jnp.float32)
    out = _decode_kernel(src3, dst3, t2, z2, bias_vec)
    return out.reshape(BATCH)
